# Initial kernel scaffold; baseline (speedup 1.0000x reference)
import jax
import jax.numpy as jnp
from jax.experimental import pallas as pl


def kernel(x, table):
    raise NotImplementedError("write your pallas kernel here")


# speedup vs baseline: 4.4336x; 4.4336x over previous
import functools

import jax
import jax.numpy as jnp
from jax import lax
from jax.experimental import pallas as pl
from jax.experimental.pallas import tpu as pltpu
from jax.experimental.pallas import tpu_sc as plsc

NC = 2
NS = 16
NW = NC * NS
L = 16


@functools.lru_cache(maxsize=None)
def _build(n, num_emb, emb_dim, nchunk):
    per_w = n // NW
    chunk = per_w // nchunk

    def body(x_hbm, table_hbm, out_hbm, table_v, idx_v, rows_v, sem):
        wid = lax.axis_index("s") * NC + lax.axis_index("c")
        base_w = wid * per_w

        pltpu.sync_copy(table_hbm, table_v)
        iota = lax.iota(jnp.int32, L)
        out_pat = [iota * emb_dim + d for d in range(emb_dim)]

        def chunk_body(c, carry):
            base = base_w + c * chunk
            pltpu.sync_copy(x_hbm.at[pl.ds(base, chunk)], idx_v)

            def vec_body(i, carry2):
                xv = idx_v[pl.ds(i * L, L)] * emb_dim
                obase = i * (L * emb_dim)
                for d in range(emb_dim):
                    g = plsc.load_gather(table_v, [xv + d])
                    plsc.store_scatter(rows_v, [out_pat[d] + obase], g)
                return carry2

            lax.fori_loop(0, chunk // L, vec_body, 0)
            pltpu.sync_copy(rows_v, out_hbm.at[pl.ds(base * emb_dim, chunk * emb_dim)])
            return carry

        lax.fori_loop(0, nchunk, chunk_body, 0)

    return pl.kernel(
        body,
        out_type=jax.ShapeDtypeStruct((n * emb_dim,), jnp.float32),
        mesh=plsc.VectorSubcoreMesh(core_axis_name="c", subcore_axis_name="s"),
        compiler_params=pltpu.CompilerParams(needs_layout_passes=False),
        scratch_types=[
            pltpu.VMEM((num_emb * emb_dim,), jnp.float32),
            pltpu.VMEM((chunk,), jnp.int32),
            pltpu.VMEM((chunk * emb_dim,), jnp.float32),
            pltpu.SemaphoreType.DMA,
        ],
    )


def kernel(x, table):
    batch, hist = x.shape
    num_emb, emb_dim = table.shape
    n = batch * hist
    fn = _build(n, num_emb, emb_dim, 4)
    out = fn(x.reshape(n).astype(jnp.int32), table.reshape(num_emb * emb_dim))
    return out.reshape(batch, hist, emb_dim)


# speedup vs baseline: 5.9350x; 1.3386x over previous
import functools

import jax
import jax.numpy as jnp
from jax import lax
from jax.experimental import pallas as pl
from jax.experimental.pallas import tpu as pltpu
from jax.experimental.pallas import tpu_sc as plsc

NC = 2
NS = 16
NW = NC * NS
L = 16


@functools.lru_cache(maxsize=None)
def _build(batch, hist, num_emb, emb_dim, rchunk):
    rows_per_w = batch // NW
    nchunk = rows_per_w // rchunk
    offs = list(range(0, hist - L + 1, L))
    if offs[-1] != hist - L:
        offs.append(hist - L)

    def body(x_hbm, table_hbm, out_hbm, table_v, idx_v, rows_v, sem):
        wid = lax.axis_index("s") * NC + lax.axis_index("c")
        base_w = wid * rows_per_w

        pltpu.sync_copy(table_hbm, table_v)
        colv = [lax.iota(jnp.int32, L) + o for o in offs]
        dval = [jnp.full((L,), d, jnp.int32) for d in range(emb_dim)]
        rval = [jnp.full((L,), r, jnp.int32) for r in range(rchunk)]

        def chunk_body(c, carry):
            r0 = base_w + c * rchunk
            pltpu.sync_copy(x_hbm.at[pl.ds(r0 * hist, rchunk * hist)], idx_v)
            for r in range(rchunk):
                for ci in range(len(offs)):
                    xv = idx_v[pl.ds(r * hist + offs[ci], L)]
                    for d in range(emb_dim):
                        g = plsc.load_gather(table_v, [xv * emb_dim + d])
                        plsc.store_scatter(rows_v, [rval[r], colv[ci], dval[d]], g)
            pltpu.sync_copy(rows_v, out_hbm.at[pl.ds(r0, rchunk)])
            return carry

        lax.fori_loop(0, nchunk, chunk_body, 0)

    return pl.kernel(
        body,
        out_type=jax.ShapeDtypeStruct((batch, hist, emb_dim), jnp.float32),
        mesh=plsc.VectorSubcoreMesh(core_axis_name="c", subcore_axis_name="s"),
        compiler_params=pltpu.CompilerParams(needs_layout_passes=False),
        scratch_types=[
            pltpu.VMEM((num_emb * emb_dim,), jnp.float32),
            pltpu.VMEM((rchunk * hist,), jnp.int32),
            pltpu.VMEM((rchunk, hist, emb_dim), jnp.float32),
            pltpu.SemaphoreType.DMA,
        ],
    )


def kernel(x, table):
    batch, hist = x.shape
    num_emb, emb_dim = table.shape
    fn = _build(batch, hist, num_emb, emb_dim, 4)
    out = fn(x.reshape(batch * hist).astype(jnp.int32),
             table.reshape(num_emb * emb_dim))
    return out


# speedup vs baseline: 39.0586x; 6.5811x over previous
import functools

import jax
import jax.numpy as jnp
from jax import lax
from jax.experimental import pallas as pl
from jax.experimental.pallas import tpu as pltpu
from jax.experimental.pallas import tpu_sc as plsc

NC = 2
NS = 16
NW = NC * NS
L = 16
BB = 128


@functools.lru_cache(maxsize=None)
def _build(batch, hist, num_emb, emb_dim, rh):
    nchunk = hist // rh
    assert nchunk * rh == hist and batch == NW * BB

    def body(xt_hbm, table_hbm, out_hbm, table_v, idx_v, rows_v, sem):
        wid = lax.axis_index("s") * NC + lax.axis_index("c")
        b0 = wid * BB
        q = wid // 2
        s0 = (wid % 2) * emb_dim

        pltpu.sync_copy(table_hbm, table_v)

        for c in range(nchunk):
            h0 = c * rh
            pltpu.sync_copy(xt_hbm.at[pl.ds(h0, rh), pl.ds(b0, BB)], idx_v)

            def row_body(r, carry):
                for cv in range(BB // L):
                    xv = idx_v[r, pl.ds(cv * L, L)] * emb_dim
                    for d in range(emb_dim):
                        rows_v[r, d, pl.ds(cv * L, L)] = plsc.load_gather(
                            table_v, [xv + d])
                return carry

            lax.fori_loop(0, rh, row_body, 0)
            pltpu.sync_copy(
                rows_v, out_hbm.at[pl.ds(h0, rh), q, pl.ds(s0, emb_dim)])

    return pl.kernel(
        body,
        out_type=jax.ShapeDtypeStruct(
            (hist, NW // 2, 2 * emb_dim, BB), jnp.float32),
        mesh=plsc.VectorSubcoreMesh(core_axis_name="c", subcore_axis_name="s"),
        compiler_params=pltpu.CompilerParams(needs_layout_passes=False),
        scratch_types=[
            pltpu.VMEM((num_emb * emb_dim,), jnp.float32),
            pltpu.VMEM((rh, BB), jnp.int32),
            pltpu.VMEM((rh, emb_dim, BB), jnp.float32),
            pltpu.SemaphoreType.DMA,
        ],
    )


def kernel(x, table):
    batch, hist = x.shape
    num_emb, emb_dim = table.shape
    fn = _build(batch, hist, num_emb, emb_dim, 40)
    out4 = fn(x.T.astype(jnp.int32), table.reshape(num_emb * emb_dim))
    out = out4.reshape(hist, NW, emb_dim, BB).transpose(1, 3, 0, 2)
    return out.reshape(batch, hist, emb_dim)


# speedup vs baseline: 69.3002x; 1.7743x over previous
import functools

import jax
import jax.numpy as jnp
from jax import lax
from jax.experimental import pallas as pl
from jax.experimental.pallas import tpu as pltpu
from jax.experimental.pallas import tpu_sc as plsc

NC = 2
NS = 16
NW = NC * NS
L = 16
BB = 128


@functools.lru_cache(maxsize=None)
def _build(batch, hist, num_emb, emb_dim, rh):
    nchunk = hist // rh
    assert nchunk * rh == hist and batch == NW * BB

    def body(xt_hbm, table_hbm, out_hbm, table_v,
             idx0, idx1, rows0, rows1, isem0, isem1, osem0, osem1):
        wid = lax.axis_index("s") * NC + lax.axis_index("c")
        b0 = wid * BB
        q = wid // 2
        s0 = (wid % 2) * emb_dim

        pltpu.sync_copy(table_hbm, table_v)
        idx = [idx0, idx1]
        rows = [rows0, rows1]
        isem = [isem0, isem1]
        osem = [osem0, osem1]

        in_cp = {}
        out_cp = {}
        in_cp[0] = pltpu.async_copy(
            xt_hbm.at[pl.ds(0, rh), pl.ds(b0, BB)], idx[0], isem[0])
        for c in range(nchunk):
            cur = c % 2
            if c + 1 < nchunk:
                in_cp[c + 1] = pltpu.async_copy(
                    xt_hbm.at[pl.ds((c + 1) * rh, rh), pl.ds(b0, BB)],
                    idx[1 - cur], isem[1 - cur])
            in_cp[c].wait()
            if c >= 2:
                out_cp[c - 2].wait()

            @plsc.parallel_loop(0, rh, unroll=4)
            def row_body(r):
                for cv in range(BB // L):
                    xv = idx[cur][r, pl.ds(cv * L, L)] * emb_dim
                    for d in range(emb_dim):
                        rows[cur][r, d, pl.ds(cv * L, L)] = plsc.load_gather(
                            table_v, [xv + d])

            out_cp[c] = pltpu.async_copy(
                rows[cur],
                out_hbm.at[pl.ds(c * rh, rh), q, pl.ds(s0, emb_dim)],
                osem[cur])
        out_cp[nchunk - 2].wait()
        out_cp[nchunk - 1].wait()

    return pl.kernel(
        body,
        out_type=jax.ShapeDtypeStruct(
            (hist, NW // 2, 2 * emb_dim, BB), jnp.float32),
        mesh=plsc.VectorSubcoreMesh(core_axis_name="c", subcore_axis_name="s"),
        compiler_params=pltpu.CompilerParams(needs_layout_passes=False),
        scratch_types=[
            pltpu.VMEM((num_emb * emb_dim,), jnp.float32),
            pltpu.VMEM((rh, BB), jnp.int32),
            pltpu.VMEM((rh, BB), jnp.int32),
            pltpu.VMEM((rh, emb_dim, BB), jnp.float32),
            pltpu.VMEM((rh, emb_dim, BB), jnp.float32),
            pltpu.SemaphoreType.DMA,
            pltpu.SemaphoreType.DMA,
            pltpu.SemaphoreType.DMA,
            pltpu.SemaphoreType.DMA,
        ],
    )


def kernel(x, table):
    batch, hist = x.shape
    num_emb, emb_dim = table.shape
    fn = _build(batch, hist, num_emb, emb_dim, 40)
    out4 = fn(x.T.astype(jnp.int32), table.reshape(num_emb * emb_dim))
    out = out4.reshape(hist, NW, emb_dim, BB).transpose(1, 3, 0, 2)
    return out.reshape(batch, hist, emb_dim)
